# baseline (speedup 1.0000x reference)
import functools

import jax
import jax.numpy as jnp
from jax import lax
from jax.experimental import pallas as pl
from jax.experimental.pallas import tpu as pltpu
from jax.experimental.pallas import tpu_sc as plsc

NV = 10000
NE = 10000
D = 128
E = 320000

NC = 2
NS = 16
NW = NC * NS

C = 64
EPW = 10240
EP = NW * EPW
NCHUNK = EPW // C
QN = NCHUNK // 4

NEP = 10240
ROWS_PER_TILE = NEP // NS
RCHUNK = 64
NRCHUNK = ROWS_PER_TILE // RCHUNK


def _sc_body(table, pk, pw, out, idxa_v, idxb_v, pwa_v, pwb_v,
             rows0, rows1, rows2, rows3, acc,
             gsem0, gsem1, gsem2, gsem3, ssem0, ssem1, ssem2, ssem3):
    c = lax.axis_index("c")
    s = lax.axis_index("s")
    wid = s * NC + c

    zero = jnp.zeros((16,), jnp.float32)

    def _zr(i, carry):
        for j in range(8):
            rows0[i, pl.ds(j * 16, 16)] = zero
        return carry

    lax.fori_loop(0, C, _zr, 0)
    row0 = s * ROWS_PER_TILE
    for k in range(NRCHUNK):
        pltpu.sync_copy(rows0, acc.at[pl.ds(row0 + k * RCHUNK, RCHUNK)])
    plsc.subcore_barrier()

    RB = ((rows0, gsem0, ssem0), (rows1, gsem1, ssem1),
          (rows2, gsem2, ssem2), (rows3, gsem3, ssem3))

    def _ldq(idx_v, pw_v, q):
        pltpu.sync_copy(pk.at[wid, pl.ds(8 * q, 8)], idx_v)
        pltpu.sync_copy(pw.at[wid, pl.ds(4 * C * q, 4 * C)], pw_v)

    def _scale(rows, pwv, toff):
        def _sb(i, cc):
            base = toff + (i & -16)
            wgrp = pwv[pl.ds(base, 16)]
            lane = jnp.full((16,), i & 15, jnp.int32)
            wb = wgrp.at[lane].get(mode="promise_in_bounds")
            for j in range(8):
                sl = pl.ds(j * 16, 16)
                rows[i, sl] = rows[i, sl] * wb
            return cc
        lax.fori_loop(0, C, _sb, 0)

    def _sg(idx_v, o, t):
        rows, gsem, _ = RB[t]
        pltpu.async_copy(table.at[idx_v.at[o]], rows, gsem)

    def _st(t, idx_v, tq, pwv):
        rows, gsem, ssem = RB[t]
        pltpu.make_async_copy(table.at[idx_v.at[2 * tq]], rows, gsem).wait()
        _scale(rows, pwv, tq * C)
        pltpu.async_copy(rows, acc.at[idx_v.at[2 * tq + 1]], ssem, add=True)

    def _free(t, idx_v):
        rows, _, ssem = RB[t]
        pltpu.make_async_copy(rows, acc.at[idx_v.at[1]], ssem).wait()

    _ldq(idxb_v, pwb_v, 0)
    _ldq(idxa_v, pwa_v, 1)
    _sg(idxb_v, 0, 0)
    _sg(idxb_v, 2, 1)
    _st(0, idxb_v, 0, pwb_v); _sg(idxb_v, 4, 2)
    _st(1, idxb_v, 1, pwb_v); _sg(idxb_v, 6, 3)
    _st(2, idxb_v, 2, pwb_v); _free(0, idxb_v); _sg(idxa_v, 0, 0)
    _st(3, idxb_v, 3, pwb_v); _free(1, idxb_v); _sg(idxa_v, 2, 1)
    _ldq(idxb_v, pwb_v, 2)

    def _body(h, carry):
        _st(0, idxa_v, 0, pwa_v); _free(2, idxa_v); _sg(idxa_v, 4, 2)
        _st(1, idxa_v, 1, pwa_v); _free(3, idxa_v); _sg(idxa_v, 6, 3)
        _st(2, idxa_v, 2, pwa_v); _free(0, idxa_v); _sg(idxb_v, 0, 0)
        _st(3, idxa_v, 3, pwa_v); _free(1, idxa_v); _sg(idxb_v, 2, 1)
        _ldq(idxa_v, pwa_v, 3 + 2 * h)
        _st(0, idxb_v, 0, pwb_v); _free(2, idxb_v); _sg(idxb_v, 4, 2)
        _st(1, idxb_v, 1, pwb_v); _free(3, idxb_v); _sg(idxb_v, 6, 3)
        _st(2, idxb_v, 2, pwb_v); _free(0, idxb_v); _sg(idxa_v, 0, 0)
        _st(3, idxb_v, 3, pwb_v); _free(1, idxb_v); _sg(idxa_v, 2, 1)
        _ldq(idxb_v, pwb_v, lax.rem(4 + 2 * h, QN))
        return carry

    lax.fori_loop(0, (NCHUNK - 8) // 8, _body, 0)

    _st(0, idxa_v, 0, pwa_v); _free(2, idxa_v); _sg(idxa_v, 4, 2)
    _st(1, idxa_v, 1, pwa_v); _free(3, idxa_v); _sg(idxa_v, 6, 3)
    _st(2, idxa_v, 2, pwa_v); _free(0, idxa_v)
    _st(3, idxa_v, 3, pwa_v); _free(1, idxa_v)
    _free(2, idxa_v)
    _free(3, idxa_v)
    plsc.subcore_barrier()

    for k in range(NRCHUNK):
        r0 = row0 + k * RCHUNK
        pltpu.sync_copy(acc.at[pl.ds(r0, RCHUNK)], rows0)
        pltpu.sync_copy(rows0, out.at[c, pl.ds(r0, RCHUNK)])


def _make_sc_scatter(interpret=False):
    mesh = plsc.VectorSubcoreMesh(core_axis_name="c", subcore_axis_name="s",
                                  num_cores=NC, num_subcores=NS)
    return pl.kernel(
        _sc_body,
        out_type=jax.ShapeDtypeStruct((NC, NEP, D), jnp.float32),
        mesh=mesh,
        scratch_types=[
            pltpu.VMEM((8, C), jnp.int32),
            pltpu.VMEM((8, C), jnp.int32),
            pltpu.VMEM((4 * C,), jnp.float32),
            pltpu.VMEM((4 * C,), jnp.float32),
            pltpu.VMEM((C, D), jnp.float32),
            pltpu.VMEM((C, D), jnp.float32),
            pltpu.VMEM((C, D), jnp.float32),
            pltpu.VMEM((C, D), jnp.float32),
            pltpu.VMEM_SHARED((NEP, D), jnp.float32),
        ] + [pltpu.SemaphoreType.DMA] * 8,
        interpret=interpret,
        name="hypermod_sc_scatter",
    )


def _pack_idx(gidx, sidx, w):
    g3 = gidx.reshape(NW, NCHUNK, 1, C)
    s3 = sidx.reshape(NW, NCHUNK, 1, C)
    pk = jnp.concatenate([g3, s3], axis=2).reshape(NW, 2 * NCHUNK, C)
    pw = w.reshape(NW, NCHUNK * C)
    return pk, pw


def _tc1_body(v_ref, vw_ref, W_ref, b_ref, ve_ref, vb_ref):
    vblk = v_ref[...]
    vw = vw_ref[...]
    ve = jnp.dot(vblk, W_ref[...], preferred_element_type=jnp.float32) + b_ref[...]
    ve_ref[...] = jnp.maximum(ve, 0.0) * vw
    vb_ref[...] = vblk * vw


def _tc2_body(e_ref, p0_ref, p1_ref, ers_ref, W_ref, b_ref, ew_ref,
              eout_ref, ev_ref):
    eacc = (e_ref[...] + p0_ref[...] + p1_ref[...]) / ers_ref[...]
    eout_ref[...] = eacc
    ev = jnp.dot(eacc, W_ref[...], preferred_element_type=jnp.float32) + b_ref[...]
    ev_ref[...] = jnp.maximum(ev, 0.0) * ew_ref[...]


def _tc3_body(vb_ref, q0_ref, q1_ref, vrs_ref, vout_ref):
    vout_ref[...] = (vb_ref[...] + q0_ref[...] + q1_ref[...]) / vrs_ref[...]


_BR = 1000
_GRID = NV // _BR

_row_blk = pl.BlockSpec((_BR, D), lambda i: (i, 0))
_sca_blk = pl.BlockSpec((_BR, 1), lambda i: (i, 0))
_W_blk = pl.BlockSpec((D, D), lambda i: (0, 0))
_b_blk = pl.BlockSpec((1, D), lambda i: (0, 0))

_tc1 = pl.pallas_call(
    _tc1_body,
    grid=(_GRID,),
    in_specs=[_row_blk, _sca_blk, _W_blk, _b_blk],
    out_specs=[_row_blk, _row_blk],
    out_shape=[jax.ShapeDtypeStruct((NV, D), jnp.float32)] * 2,
)

_tc2 = pl.pallas_call(
    _tc2_body,
    grid=(_GRID,),
    in_specs=[_row_blk, _row_blk, _row_blk, _sca_blk, _W_blk, _b_blk, _sca_blk],
    out_specs=[_row_blk, _row_blk],
    out_shape=[jax.ShapeDtypeStruct((NE, D), jnp.float32)] * 2,
)

_tc3 = pl.pallas_call(
    _tc3_body,
    grid=(_GRID,),
    in_specs=[_row_blk, _row_blk, _row_blk, _sca_blk],
    out_specs=_row_blk,
    out_shape=jax.ShapeDtypeStruct((NV, D), jnp.float32),
)


def kernel(v, e, vidx, eidx, ver2edg, v_weight, e_weight, v_reg_weight,
           e_reg_weight, v_reg_sum, e_reg_sum, W_v2e, W_e2v, b_v, b_e):
    pad = EP - E
    padi = jnp.zeros((pad,), jnp.int32)
    padf = jnp.zeros((pad,), jnp.float32)
    vidx = jnp.concatenate([vidx.astype(jnp.int32), padi])
    eidx = jnp.concatenate([eidx.astype(jnp.int32), padi])
    w1 = jnp.concatenate([v_reg_weight[:, 0], padf])
    w2 = jnp.concatenate([e_reg_weight[:, 0], padf])

    sc_scatter = _make_sc_scatter()

    pk1, pw1 = _pack_idx(vidx, eidx, w1)
    pk2, pw2 = _pack_idx(eidx, vidx, w2)

    ve, v_base = _tc1(v, v_weight, W_v2e, b_v.reshape(1, D))
    parts_e = sc_scatter(ve, pk1, pw1)
    e_out, ev = _tc2(e, parts_e[0], parts_e[1], e_reg_sum, W_e2v,
                     b_e.reshape(1, D), e_weight)
    parts_v = sc_scatter(ev, pk2, pw2)
    v_out = _tc3(v_base, parts_v[0], parts_v[1], v_reg_sum)
    return (v_out, e_out)


# speedup vs baseline: 1.0335x; 1.0335x over previous
import functools

import jax
import jax.numpy as jnp
from jax import lax
from jax.experimental import pallas as pl
from jax.experimental.pallas import tpu as pltpu
from jax.experimental.pallas import tpu_sc as plsc

NV = 10000
NE = 10000
D = 128
E = 320000

NC = 2
NS = 16
NW = NC * NS

C = 64
EPW = 10240
EP = NW * EPW
NCHUNK = EPW // C
QN = NCHUNK // 4

NEP = 10240
ROWS_PER_TILE = NEP // NS
RCHUNK = 64
NRCHUNK = ROWS_PER_TILE // RCHUNK


def _sc_body(table, pk, pw, out, idxa_v, idxb_v, pwa_v, pwb_v,
             rows0, rows1, rows2, rows3, acc,
             gsem0, gsem1, gsem2, gsem3, ssem0, ssem1, ssem2, ssem3):
    c = lax.axis_index("c")
    s = lax.axis_index("s")
    wid = s * NC + c

    zero = jnp.zeros((16,), jnp.float32)

    def _zr(i, carry):
        for j in range(8):
            rows0[i, pl.ds(j * 16, 16)] = zero
        return carry

    lax.fori_loop(0, C, _zr, 0)
    row0 = s * ROWS_PER_TILE
    for k in range(NRCHUNK):
        pltpu.sync_copy(rows0, acc.at[pl.ds(row0 + k * RCHUNK, RCHUNK)])
    plsc.subcore_barrier()

    RB = ((rows0, gsem0, ssem0), (rows1, gsem1, ssem1),
          (rows2, gsem2, ssem2), (rows3, gsem3, ssem3))

    def _ldq(idx_v, pw_v, q):
        pltpu.sync_copy(pk.at[wid, pl.ds(8 * q, 8)], idx_v)
        pltpu.sync_copy(pw.at[wid, pl.ds(4 * C * q, 4 * C)], pw_v)

    def _scale(rows, pwv, toff):
        def _sb(g, cc):
            wgrp = pwv[pl.ds(toff + g * 16, 16)]
            for l in range(16):
                wb = wgrp.at[jnp.full((16,), l, jnp.int32)].get(
                    mode="promise_in_bounds")
                r = g * 16 + l
                for j in range(8):
                    sl = pl.ds(j * 16, 16)
                    rows[r, sl] = rows[r, sl] * wb
            return cc
        lax.fori_loop(0, C // 16, _sb, 0)

    def _sg(idx_v, o, t):
        rows, gsem, _ = RB[t]
        pltpu.async_copy(table.at[idx_v.at[o]], rows, gsem)

    def _st(t, idx_v, tq, pwv):
        rows, gsem, ssem = RB[t]
        pltpu.make_async_copy(table.at[idx_v.at[2 * tq]], rows, gsem).wait()
        _scale(rows, pwv, tq * C)
        pltpu.async_copy(rows, acc.at[idx_v.at[2 * tq + 1]], ssem, add=True)

    def _free(t, idx_v):
        rows, _, ssem = RB[t]
        pltpu.make_async_copy(rows, acc.at[idx_v.at[1]], ssem).wait()

    _ldq(idxb_v, pwb_v, 0)
    _ldq(idxa_v, pwa_v, 1)
    _sg(idxb_v, 0, 0)
    _sg(idxb_v, 2, 1)
    _st(0, idxb_v, 0, pwb_v); _sg(idxb_v, 4, 2)
    _st(1, idxb_v, 1, pwb_v); _sg(idxb_v, 6, 3)
    _st(2, idxb_v, 2, pwb_v); _free(0, idxb_v); _sg(idxa_v, 0, 0)
    _st(3, idxb_v, 3, pwb_v); _free(1, idxb_v); _sg(idxa_v, 2, 1)
    _ldq(idxb_v, pwb_v, 2)

    def _body(h, carry):
        _st(0, idxa_v, 0, pwa_v); _free(2, idxa_v); _sg(idxa_v, 4, 2)
        _st(1, idxa_v, 1, pwa_v); _free(3, idxa_v); _sg(idxa_v, 6, 3)
        _st(2, idxa_v, 2, pwa_v); _free(0, idxa_v); _sg(idxb_v, 0, 0)
        _st(3, idxa_v, 3, pwa_v); _free(1, idxa_v); _sg(idxb_v, 2, 1)
        _ldq(idxa_v, pwa_v, 3 + 2 * h)
        _st(0, idxb_v, 0, pwb_v); _free(2, idxb_v); _sg(idxb_v, 4, 2)
        _st(1, idxb_v, 1, pwb_v); _free(3, idxb_v); _sg(idxb_v, 6, 3)
        _st(2, idxb_v, 2, pwb_v); _free(0, idxb_v); _sg(idxa_v, 0, 0)
        _st(3, idxb_v, 3, pwb_v); _free(1, idxb_v); _sg(idxa_v, 2, 1)
        _ldq(idxb_v, pwb_v, lax.rem(4 + 2 * h, QN))
        return carry

    lax.fori_loop(0, (NCHUNK - 8) // 8, _body, 0)

    _st(0, idxa_v, 0, pwa_v); _free(2, idxa_v); _sg(idxa_v, 4, 2)
    _st(1, idxa_v, 1, pwa_v); _free(3, idxa_v); _sg(idxa_v, 6, 3)
    _st(2, idxa_v, 2, pwa_v); _free(0, idxa_v)
    _st(3, idxa_v, 3, pwa_v); _free(1, idxa_v)
    _free(2, idxa_v)
    _free(3, idxa_v)
    plsc.subcore_barrier()

    for k in range(NRCHUNK):
        r0 = row0 + k * RCHUNK
        pltpu.sync_copy(acc.at[pl.ds(r0, RCHUNK)], rows0)
        pltpu.sync_copy(rows0, out.at[c, pl.ds(r0, RCHUNK)])


def _make_sc_scatter(interpret=False):
    mesh = plsc.VectorSubcoreMesh(core_axis_name="c", subcore_axis_name="s",
                                  num_cores=NC, num_subcores=NS)
    return pl.kernel(
        _sc_body,
        out_type=jax.ShapeDtypeStruct((NC, NEP, D), jnp.float32),
        mesh=mesh,
        scratch_types=[
            pltpu.VMEM((8, C), jnp.int32),
            pltpu.VMEM((8, C), jnp.int32),
            pltpu.VMEM((4 * C,), jnp.float32),
            pltpu.VMEM((4 * C,), jnp.float32),
            pltpu.VMEM((C, D), jnp.float32),
            pltpu.VMEM((C, D), jnp.float32),
            pltpu.VMEM((C, D), jnp.float32),
            pltpu.VMEM((C, D), jnp.float32),
            pltpu.VMEM_SHARED((NEP, D), jnp.float32),
        ] + [pltpu.SemaphoreType.DMA] * 8,
        interpret=interpret,
        name="hypermod_sc_scatter",
    )


def _pack_idx(gidx, sidx, w):
    g3 = gidx.reshape(NW, NCHUNK, 1, C)
    s3 = sidx.reshape(NW, NCHUNK, 1, C)
    pk = jnp.concatenate([g3, s3], axis=2).reshape(NW, 2 * NCHUNK, C)
    pw = w.reshape(NW, NCHUNK * C)
    return pk, pw


def _tc1_body(v_ref, vw_ref, W_ref, b_ref, ve_ref, vb_ref):
    vblk = v_ref[...]
    vw = vw_ref[...]
    ve = jnp.dot(vblk, W_ref[...], preferred_element_type=jnp.float32) + b_ref[...]
    ve_ref[...] = jnp.maximum(ve, 0.0) * vw
    vb_ref[...] = vblk * vw


def _tc2_body(e_ref, p0_ref, p1_ref, ers_ref, W_ref, b_ref, ew_ref,
              eout_ref, ev_ref):
    eacc = (e_ref[...] + p0_ref[...] + p1_ref[...]) / ers_ref[...]
    eout_ref[...] = eacc
    ev = jnp.dot(eacc, W_ref[...], preferred_element_type=jnp.float32) + b_ref[...]
    ev_ref[...] = jnp.maximum(ev, 0.0) * ew_ref[...]


def _tc3_body(vb_ref, q0_ref, q1_ref, vrs_ref, vout_ref):
    vout_ref[...] = (vb_ref[...] + q0_ref[...] + q1_ref[...]) / vrs_ref[...]


_BR = 1000
_GRID = NV // _BR

_row_blk = pl.BlockSpec((_BR, D), lambda i: (i, 0))
_sca_blk = pl.BlockSpec((_BR, 1), lambda i: (i, 0))
_W_blk = pl.BlockSpec((D, D), lambda i: (0, 0))
_b_blk = pl.BlockSpec((1, D), lambda i: (0, 0))

_tc1 = pl.pallas_call(
    _tc1_body,
    grid=(_GRID,),
    in_specs=[_row_blk, _sca_blk, _W_blk, _b_blk],
    out_specs=[_row_blk, _row_blk],
    out_shape=[jax.ShapeDtypeStruct((NV, D), jnp.float32)] * 2,
)

_tc2 = pl.pallas_call(
    _tc2_body,
    grid=(_GRID,),
    in_specs=[_row_blk, _row_blk, _row_blk, _sca_blk, _W_blk, _b_blk, _sca_blk],
    out_specs=[_row_blk, _row_blk],
    out_shape=[jax.ShapeDtypeStruct((NE, D), jnp.float32)] * 2,
)

_tc3 = pl.pallas_call(
    _tc3_body,
    grid=(_GRID,),
    in_specs=[_row_blk, _row_blk, _row_blk, _sca_blk],
    out_specs=_row_blk,
    out_shape=jax.ShapeDtypeStruct((NV, D), jnp.float32),
)


def kernel(v, e, vidx, eidx, ver2edg, v_weight, e_weight, v_reg_weight,
           e_reg_weight, v_reg_sum, e_reg_sum, W_v2e, W_e2v, b_v, b_e):
    pad = EP - E
    padi = jnp.zeros((pad,), jnp.int32)
    padf = jnp.zeros((pad,), jnp.float32)
    vidx = jnp.concatenate([vidx.astype(jnp.int32), padi])
    eidx = jnp.concatenate([eidx.astype(jnp.int32), padi])
    w1 = jnp.concatenate([v_reg_weight[:, 0], padf])
    w2 = jnp.concatenate([e_reg_weight[:, 0], padf])

    sc_scatter = _make_sc_scatter()

    pk1, pw1 = _pack_idx(vidx, eidx, w1)
    pk2, pw2 = _pack_idx(eidx, vidx, w2)

    ve, v_base = _tc1(v, v_weight, W_v2e, b_v.reshape(1, D))
    parts_e = sc_scatter(ve, pk1, pw1)
    e_out, ev = _tc2(e, parts_e[0], parts_e[1], e_reg_sum, W_e2v,
                     b_e.reshape(1, D), e_weight)
    parts_v = sc_scatter(ev, pk2, pw2)
    v_out = _tc3(v_base, parts_v[0], parts_v[1], v_reg_sum)
    return (v_out, e_out)


# speedup vs baseline: 1.0636x; 1.0292x over previous
import functools

import jax
import jax.numpy as jnp
from jax import lax
from jax.experimental import pallas as pl
from jax.experimental.pallas import tpu as pltpu
from jax.experimental.pallas import tpu_sc as plsc

NV = 10000
NE = 10000
D = 128
E = 320000

NC = 2
NS = 16
NW = NC * NS

C = 128
EPW = 10240
EP = NW * EPW
NCHUNK = EPW // C

NEP = 10240
ROWS_PER_TILE = NEP // NS
RCHUNK = 128
NRCHUNK = ROWS_PER_TILE // RCHUNK


def _sc_body(table, pk, pw, out, idxa_v, idxb_v, pw_v, rows0, rows1, acc,
             sem0, sem1):
    c = lax.axis_index("c")
    s = lax.axis_index("s")
    wid = s * NC + c

    pltpu.sync_copy(pw.at[wid], pw_v)

    zero = jnp.zeros((16,), jnp.float32)

    def _zr(i, carry):
        for j in range(8):
            rows0[i, pl.ds(j * 16, 16)] = zero
        return carry

    lax.fori_loop(0, C, _zr, 0)
    row0 = s * ROWS_PER_TILE
    for k in range(NRCHUNK):
        pltpu.sync_copy(rows0, acc.at[pl.ds(row0 + k * RCHUNK, RCHUNK)])
    plsc.subcore_barrier()

    def _process(chunk, rows):
        def _scale(g, cc):
            wgrp = pw_v[chunk, pl.ds(g * 16, 16)]
            for l in range(16):
                wb = wgrp.at[jnp.full((16,), l, jnp.int32)].get(
                    mode="promise_in_bounds")
                r = g * 16 + l
                for j in range(8):
                    sl = pl.ds(j * 16, 16)
                    rows[r, sl] = rows[r, sl] * wb
            return cc

        lax.fori_loop(0, C // 16, _scale, 0)

    QN = NCHUNK // 4

    def _ldq(idx_v, q):
        pltpu.sync_copy(pk.at[wid, pl.ds(8 * q, 8)], idx_v)

    def _step(chunk, idx_v, o, rows, sem, nidx_v, no):
        pltpu.make_async_copy(table.at[idx_v.at[o]], rows, sem).wait()
        _process(chunk, rows)
        pltpu.sync_copy(rows, acc.at[idx_v.at[o + 1]], add=True)
        pltpu.async_copy(table.at[nidx_v.at[no]], rows, sem)

    _ldq(idxa_v, 0)
    _ldq(idxb_v, 1)
    pltpu.async_copy(table.at[idxa_v.at[0]], rows0, sem0)
    pltpu.async_copy(table.at[idxa_v.at[2]], rows1, sem1)

    def _oct(g, carry):
        k0 = 8 * g
        _step(k0 + 0, idxa_v, 0, rows0, sem0, idxa_v, 4)
        _step(k0 + 1, idxa_v, 2, rows1, sem1, idxa_v, 6)
        _step(k0 + 2, idxa_v, 4, rows0, sem0, idxb_v, 0)
        _step(k0 + 3, idxa_v, 6, rows1, sem1, idxb_v, 2)
        _ldq(idxa_v, lax.rem(2 * g + 2, QN))
        _step(k0 + 4, idxb_v, 0, rows0, sem0, idxb_v, 4)
        _step(k0 + 5, idxb_v, 2, rows1, sem1, idxb_v, 6)
        _step(k0 + 6, idxb_v, 4, rows0, sem0, idxa_v, 0)
        _step(k0 + 7, idxb_v, 6, rows1, sem1, idxa_v, 2)
        _ldq(idxb_v, lax.rem(2 * g + 3, QN))
        return carry

    lax.fori_loop(0, NCHUNK // 8, _oct, 0)
    pltpu.make_async_copy(table.at[idxa_v.at[0]], rows0, sem0).wait()
    pltpu.make_async_copy(table.at[idxa_v.at[2]], rows1, sem1).wait()
    plsc.subcore_barrier()

    for k in range(NRCHUNK):
        r0 = row0 + k * RCHUNK
        pltpu.sync_copy(acc.at[pl.ds(r0, RCHUNK)], rows0)
        pltpu.sync_copy(rows0, out.at[c, pl.ds(r0, RCHUNK)])


def _make_sc_scatter(interpret=False):
    mesh = plsc.VectorSubcoreMesh(core_axis_name="c", subcore_axis_name="s",
                                  num_cores=NC, num_subcores=NS)
    return pl.kernel(
        _sc_body,
        out_type=jax.ShapeDtypeStruct((NC, NEP, D), jnp.float32),
        mesh=mesh,
        scratch_types=[
            pltpu.VMEM((8, C), jnp.int32),
            pltpu.VMEM((8, C), jnp.int32),
            pltpu.VMEM((NCHUNK, C), jnp.float32),
            pltpu.VMEM((C, D), jnp.float32),
            pltpu.VMEM((C, D), jnp.float32),
            pltpu.VMEM_SHARED((NEP, D), jnp.float32),
            pltpu.SemaphoreType.DMA,
            pltpu.SemaphoreType.DMA,
        ],
        interpret=interpret,
        name="hypermod_sc_scatter",
    )


def _pack_idx(gidx, sidx, w):
    g3 = gidx.reshape(NW, NCHUNK, 1, C)
    s3 = sidx.reshape(NW, NCHUNK, 1, C)
    pk = jnp.concatenate([g3, s3], axis=2).reshape(NW, 2 * NCHUNK, C)
    pw = w.reshape(NW, NCHUNK, C)
    return pk, pw


def _tc1_body(v_ref, vw_ref, W_ref, b_ref, ve_ref, vb_ref):
    vblk = v_ref[...]
    vw = vw_ref[...]
    ve = jnp.dot(vblk, W_ref[...], preferred_element_type=jnp.float32) + b_ref[...]
    ve_ref[...] = jnp.maximum(ve, 0.0) * vw
    vb_ref[...] = vblk * vw


def _tc2_body(e_ref, p0_ref, p1_ref, ers_ref, W_ref, b_ref, ew_ref,
              eout_ref, ev_ref):
    eacc = (e_ref[...] + p0_ref[...] + p1_ref[...]) / ers_ref[...]
    eout_ref[...] = eacc
    ev = jnp.dot(eacc, W_ref[...], preferred_element_type=jnp.float32) + b_ref[...]
    ev_ref[...] = jnp.maximum(ev, 0.0) * ew_ref[...]


def _tc3_body(vb_ref, q0_ref, q1_ref, vrs_ref, vout_ref):
    vout_ref[...] = (vb_ref[...] + q0_ref[...] + q1_ref[...]) / vrs_ref[...]


_BR = 1000
_GRID = NV // _BR

_row_blk = pl.BlockSpec((_BR, D), lambda i: (i, 0))
_sca_blk = pl.BlockSpec((_BR, 1), lambda i: (i, 0))
_W_blk = pl.BlockSpec((D, D), lambda i: (0, 0))
_b_blk = pl.BlockSpec((1, D), lambda i: (0, 0))

_tc1 = pl.pallas_call(
    _tc1_body,
    grid=(_GRID,),
    in_specs=[_row_blk, _sca_blk, _W_blk, _b_blk],
    out_specs=[_row_blk, _row_blk],
    out_shape=[jax.ShapeDtypeStruct((NV, D), jnp.float32)] * 2,
)

_tc2 = pl.pallas_call(
    _tc2_body,
    grid=(_GRID,),
    in_specs=[_row_blk, _row_blk, _row_blk, _sca_blk, _W_blk, _b_blk, _sca_blk],
    out_specs=[_row_blk, _row_blk],
    out_shape=[jax.ShapeDtypeStruct((NE, D), jnp.float32)] * 2,
)

_tc3 = pl.pallas_call(
    _tc3_body,
    grid=(_GRID,),
    in_specs=[_row_blk, _row_blk, _row_blk, _sca_blk],
    out_specs=_row_blk,
    out_shape=jax.ShapeDtypeStruct((NV, D), jnp.float32),
)


def kernel(v, e, vidx, eidx, ver2edg, v_weight, e_weight, v_reg_weight,
           e_reg_weight, v_reg_sum, e_reg_sum, W_v2e, W_e2v, b_v, b_e):
    pad = EP - E
    padi = jnp.zeros((pad,), jnp.int32)
    padf = jnp.zeros((pad,), jnp.float32)
    vidx = jnp.concatenate([vidx.astype(jnp.int32), padi])
    eidx = jnp.concatenate([eidx.astype(jnp.int32), padi])
    w1 = jnp.concatenate([v_reg_weight[:, 0], padf])
    w2 = jnp.concatenate([e_reg_weight[:, 0], padf])

    sc_scatter = _make_sc_scatter()

    pk1, pw1 = _pack_idx(vidx, eidx, w1)
    pk2, pw2 = _pack_idx(eidx, vidx, w2)

    ve, v_base = _tc1(v, v_weight, W_v2e, b_v.reshape(1, D))
    parts_e = sc_scatter(ve, pk1, pw1)
    e_out, ev = _tc2(e, parts_e[0], parts_e[1], e_reg_sum, W_e2v,
                     b_e.reshape(1, D), e_weight)
    parts_v = sc_scatter(ev, pk2, pw2)
    v_out = _tc3(v_base, parts_v[0], parts_v[1], v_reg_sum)
    return (v_out, e_out)
